# baseline (speedup 1.0000x reference)
import jax
import jax.numpy as jnp
from jax import lax
from jax.experimental import pallas as pl
from jax.experimental.pallas import tpu as pltpu
from jax.experimental.pallas import tpu_sc as plsc

B = 2
S = 128
C = 4
G = 64
N = B * G * G * G
NC, NS, L = 2, 16, 16
NW = NC * NS
PER_TILE = N // NW
P = 1024
NCHUNK = PER_TILE // P
IDXN = 128

_OFFS = [dx * S * S + dy * S + dz
         for dx in (0, 1) for dy in (0, 1) for dz in (0, 1)]


def _sc_body(table_hbm, cx_hbm, cy_hbm, cz_hbm, out_hbm, *scr):
    (cxv, cyv, czv, fxv, fyv, fzv, idxv, rowsv) = (
        [scr[i], scr[i]] for i in range(8))
    outv, sem_c = scr[8], scr[9]
    sem_g = [scr[10], scr[10]]

    wid = lax.axis_index("s") * NC + lax.axis_index("c")
    tile_base = wid * PER_TILE
    lane = lax.iota(jnp.int32, L)

    def prep(start, a):
        @pl.loop(0, P, step=L)
        def _prep(i):
            x = cxv[a][pl.ds(i, L)]
            y = cyv[a][pl.ds(i, L)]
            z = czv[a][pl.ds(i, L)]
            ix = jnp.clip(x.astype(jnp.int32), 0, S - 2)
            iy = jnp.clip(y.astype(jnp.int32), 0, S - 2)
            iz = jnp.clip(z.astype(jnp.int32), 0, S - 2)
            fxv[a][pl.ds(i, L)] = jnp.clip(x - ix.astype(jnp.float32), 0., 1.)
            fyv[a][pl.ds(i, L)] = jnp.clip(y - iy.astype(jnp.float32), 0., 1.)
            fzv[a][pl.ds(i, L)] = jnp.clip(z - iz.astype(jnp.float32), 0., 1.)
            b = (start + i + lane) >> 18
            vox = (b << 21) + (ix << 14) + (iy << 7) + iz
            for c in range(8):
                idxv[a][pl.ds(c * P + i, L)] = vox + _OFFS[c]

    def fire_gathers(a):
        @pl.loop(0, 8 * P, step=IDXN)
        def _g(k):
            pltpu.async_copy(
                table_hbm.at[idxv[a].at[pl.ds(k, IDXN)]],
                rowsv[a].at[pl.ds(k, IDXN)], sem_g[a]).wait()

    def blend(start, a):
        @pl.loop(0, P, step=L)
        def _blend(j):
            fx = fxv[a][pl.ds(j, L)]
            fy = fyv[a][pl.ds(j, L)]
            fz = fzv[a][pl.ds(j, L)]
            row = j + lane
            for ch in range(C):
                col = jnp.full((L,), ch, jnp.int32)
                s = [plsc.load_gather(rowsv[a], [c * P + row, col])
                     for c in range(8)]
                a00 = s[0] + fz * (s[1] - s[0])
                a01 = s[2] + fz * (s[3] - s[2])
                a10 = s[4] + fz * (s[5] - s[4])
                a11 = s[6] + fz * (s[7] - s[6])
                b0 = a00 + fy * (a01 - a00)
                b1 = a10 + fy * (a11 - a10)
                fi = (row << 2) + ch
                plsc.store_scatter(outv, [fi >> 7, fi & 127],
                                   b0 + fx * (b1 - b0))
        pltpu.sync_copy(outv, out_hbm.at[pl.ds(start >> 5, P * C // 128)])

    @pl.loop(0, NCHUNK)
    def _chunk(c):
        start = tile_base + c * P
        pltpu.sync_copy(cx_hbm.at[pl.ds(start, P)], cxv[0])
        pltpu.sync_copy(cy_hbm.at[pl.ds(start, P)], cyv[0])
        pltpu.sync_copy(cz_hbm.at[pl.ds(start, P)], czv[0])
        prep(start, 0)
        fire_gathers(0)
        blend(start, 0)


@jax.jit
def kernel(inputs, sample_coords):
    table = inputs.reshape(B * S * S * S, C)
    coords = sample_coords.reshape(N, 3)
    cx = coords[:, 0]
    cy = coords[:, 1]
    cz = coords[:, 2]

    cp = pltpu.CompilerParams(
        needs_layout_passes=False, use_tc_tiling_on_sc=False)
    mesh = plsc.VectorSubcoreMesh(core_axis_name="c", subcore_axis_name="s")
    run = pl.kernel(
        _sc_body,
        out_type=jax.ShapeDtypeStruct((N * C // 128, 128), jnp.float32),
        mesh=mesh,
        scratch_types=[
            pltpu.VMEM((P,), jnp.float32),
            pltpu.VMEM((P,), jnp.float32),
            pltpu.VMEM((P,), jnp.float32),
            pltpu.VMEM((P,), jnp.float32),
            pltpu.VMEM((P,), jnp.float32),
            pltpu.VMEM((P,), jnp.float32),
            pltpu.VMEM((8 * P,), jnp.int32),
            pltpu.VMEM((8 * P, C), jnp.float32),
            pltpu.VMEM((P * C // 128, 128), jnp.float32),
            pltpu.SemaphoreType.DMA,
            pltpu.SemaphoreType.DMA,
        ],
        compiler_params=cp,
    )
    out = run(table, cx, cy, cz)
    return out.reshape(B, G, G, G, C)


# speedup vs baseline: 1.0509x; 1.0509x over previous
import jax
import jax.numpy as jnp
from jax import lax
from jax.experimental import pallas as pl
from jax.experimental.pallas import tpu as pltpu
from jax.experimental.pallas import tpu_sc as plsc

B = 2
S = 128
C = 4
G = 64
N = B * G * G * G
NC, NS, L = 2, 16, 16
NW = NC * NS
PER_TILE = N // NW
P = 1024
NCHUNK = PER_TILE // P
IDXN = 128

_OFFS = [dx * S * S + dy * S + dz
         for dx in (0, 1) for dy in (0, 1) for dz in (0, 1)]


def _sc_body(table_hbm, cx_hbm, cy_hbm, cz_hbm, out_hbm, *scr):
    (cxv, cyv, czv, fxv, fyv, fzv, idxv, rowsv) = (
        [scr[i], scr[i]] for i in range(8))
    outv, sem_c = scr[8], scr[9]
    sem_g = [scr[10], scr[10]]

    wid = lax.axis_index("s") * NC + lax.axis_index("c")
    tile_base = wid * PER_TILE
    lane = lax.iota(jnp.int32, L)

    def prep(start, a):
        @pl.loop(0, P, step=L)
        def _prep(i):
            x = cxv[a][pl.ds(i, L)]
            y = cyv[a][pl.ds(i, L)]
            z = czv[a][pl.ds(i, L)]
            ix = jnp.clip(x.astype(jnp.int32), 0, S - 2)
            iy = jnp.clip(y.astype(jnp.int32), 0, S - 2)
            iz = jnp.clip(z.astype(jnp.int32), 0, S - 2)
            fxv[a][pl.ds(i, L)] = jnp.clip(x - ix.astype(jnp.float32), 0., 1.)
            fyv[a][pl.ds(i, L)] = jnp.clip(y - iy.astype(jnp.float32), 0., 1.)
            fzv[a][pl.ds(i, L)] = jnp.clip(z - iz.astype(jnp.float32), 0., 1.)
            b = (start + i + lane) >> 18
            vox = (b << 21) + (ix << 14) + (iy << 7) + iz
            for c in range(8):
                idxv[a][pl.ds(c * P + i, L)] = vox + _OFFS[c]

    def fire_gathers(a):
        @pl.loop(0, 8 * P, step=IDXN)
        def _g(k):
            pltpu.async_copy(
                table_hbm.at[idxv[a].at[pl.ds(k, IDXN)]],
                rowsv[a].at[pl.ds(k, IDXN)], sem_g[a]).wait()

    def blend(start, a):
        @pl.loop(0, P, step=L)
        def _blend(j):
            fx = fxv[a][pl.ds(j, L)]
            fy = fyv[a][pl.ds(j, L)]
            fz = fzv[a][pl.ds(j, L)]
            row = j + lane
            for ch in range(C):
                col = jnp.full((L,), ch, jnp.int32)
                s = [plsc.load_gather(rowsv[a], [c * P + row, col])
                     for c in range(8)]
                a00 = s[0] + fz * (s[1] - s[0])
                a01 = s[2] + fz * (s[3] - s[2])
                a10 = s[4] + fz * (s[5] - s[4])
                a11 = s[6] + fz * (s[7] - s[6])
                b0 = a00 + fy * (a01 - a00)
                b1 = a10 + fy * (a11 - a10)
                fi = ((row >> 6) << 9) + (ch << 7) + (row & 63)
                plsc.store_scatter(outv, [fi >> 9, fi & 511],
                                   b0 + fx * (b1 - b0))
        pltpu.sync_copy(outv, out_hbm.at[pl.ds(start >> 6, P >> 6)])

    @pl.loop(0, NCHUNK)
    def _chunk(c):
        start = tile_base + c * P
        pltpu.sync_copy(cx_hbm.at[pl.ds(start, P)], cxv[0])
        pltpu.sync_copy(cy_hbm.at[pl.ds(start, P)], cyv[0])
        pltpu.sync_copy(cz_hbm.at[pl.ds(start, P)], czv[0])
        prep(start, 0)
        fire_gathers(0)
        blend(start, 0)


@jax.jit
def kernel(inputs, sample_coords):
    table = inputs.reshape(B * S * S * S, C)
    coords = sample_coords.reshape(N, 3)
    cx = coords[:, 0]
    cy = coords[:, 1]
    cz = coords[:, 2]

    cp = pltpu.CompilerParams(
        needs_layout_passes=False, use_tc_tiling_on_sc=False)
    mesh = plsc.VectorSubcoreMesh(core_axis_name="c", subcore_axis_name="s")
    run = pl.kernel(
        _sc_body,
        out_type=jax.ShapeDtypeStruct((N // G, C * 128), jnp.float32),
        mesh=mesh,
        scratch_types=[
            pltpu.VMEM((P,), jnp.float32),
            pltpu.VMEM((P,), jnp.float32),
            pltpu.VMEM((P,), jnp.float32),
            pltpu.VMEM((P,), jnp.float32),
            pltpu.VMEM((P,), jnp.float32),
            pltpu.VMEM((P,), jnp.float32),
            pltpu.VMEM((8 * P,), jnp.int32),
            pltpu.VMEM((8 * P, C), jnp.float32),
            pltpu.VMEM((P // G, C * 128), jnp.float32),
            pltpu.SemaphoreType.DMA,
            pltpu.SemaphoreType.DMA,
        ],
        compiler_params=cp,
    )
    out5 = run(table, cx, cy, cz).reshape(B, G, G, C, 128)
    return jnp.swapaxes(out5, 3, 4)[:, :, :, :G, :]
